# baseline (speedup 1.0000x reference)
import functools
import math

import jax
import jax.numpy as jnp
from jax import lax
from jax.experimental import pallas as pl
from jax.experimental.pallas import tpu as pltpu
from jax.experimental.pallas import tpu_sc as plsc


def _gather_rows_sc(table, idx):
    n_rows = idx.shape[0]
    _, d = table.shape
    info = plsc.get_sparse_core_info()
    nw = info.num_cores * info.num_subcores
    b_per_w = n_rows // nw
    mesh = plsc.VectorSubcoreMesh(core_axis_name="c", subcore_axis_name="s")

    @functools.partial(
        pl.kernel,
        mesh=mesh,
        out_type=jax.ShapeDtypeStruct((n_rows, d), table.dtype),
        scratch_types=[
            pltpu.VMEM((b_per_w,), jnp.int32),
            pltpu.VMEM((b_per_w, d), table.dtype),
            pltpu.SemaphoreType.DMA,
        ],
    )
    def gather_kernel(table_hbm, idx_hbm, out_hbm, idx_v, rows_v, sem):
        wid = lax.axis_index("s") * info.num_cores + lax.axis_index("c")
        base = wid * b_per_w
        pltpu.sync_copy(idx_hbm.at[pl.ds(base, b_per_w)], idx_v)
        pltpu.async_copy(table_hbm.at[idx_v], rows_v, sem).wait()
        pltpu.sync_copy(rows_v, out_hbm.at[pl.ds(base, b_per_w)])

    return gather_kernel(table, idx)


def _matmul_body(x_ref, pos_ref, w_ref, b_ref, o_ref, *, scale):
    xb = (x_ref[...] * scale + pos_ref[0]).astype(jnp.bfloat16)
    acc = lax.dot_general(
        xb, w_ref[...], (((1,), (1,)), ((), ())),
        preferred_element_type=jnp.float32,
    )
    o_ref[...] = acc + b_ref[...]


def kernel(src, tgt, emb, pos_src, pos_tgt, Wc, bc):
    b, l = src.shape
    v, d = emb.shape
    m = 2 * b * l
    bm = l

    idx = jnp.concatenate([src.reshape(-1), tgt.reshape(-1)]).astype(jnp.int32)
    x = _gather_rows_sc(emb, idx)

    pos_cat = jnp.stack([pos_src[:l, :d], pos_tgt[:l, :d]])
    w_bf = Wc.astype(jnp.bfloat16)
    bc2 = bc.reshape(1, v)
    groups_per_s = (b * l) // bm

    out = pl.pallas_call(
        functools.partial(_matmul_body, scale=math.sqrt(d)),
        grid=(m // bm,),
        in_specs=[
            pl.BlockSpec((bm, d), lambda i: (i, 0)),
            pl.BlockSpec((1, bm, d), lambda i: (i // groups_per_s, 0, 0)),
            pl.BlockSpec((v, d), lambda i: (0, 0)),
            pl.BlockSpec((1, v), lambda i: (0, 0)),
        ],
        out_specs=pl.BlockSpec((bm, v), lambda i: (i, 0)),
        out_shape=jax.ShapeDtypeStruct((m, v), jnp.float32),
        compiler_params=pltpu.CompilerParams(
            dimension_semantics=("arbitrary",),
        ),
    )(x, pos_cat, w_bf, bc2)

    return out.reshape(2, b, l, v)
